# baseline (speedup 1.0000x reference)
import jax
import jax.numpy as jnp
from jax import lax
from jax.experimental import pallas as pl
from jax.experimental.pallas import tpu as pltpu
from jax.experimental.pallas import tpu_sc as plsc

N, S = 1024, 8192
L = 16
NC, NS = 2, 16
NW = NC * NS
ROWS_PER_W = N // NW
NQ = N // 4
UNROLL = 8
GROUPS = S // (UNROLL * L)

_GDN = lax.GatherDimensionNumbers(
    offset_dims=(), collapsed_slice_dims=(0,), start_index_map=(0,))


def _gather16(v, idx):
    return lax.gather(v, idx[:, None], _GDN, slice_sizes=(1,),
                      mode=lax.GatherScatterMode.PROMISE_IN_BOUNDS)


def _ffill_body(attn_hbm, mask_hbm, out_hbm, attn_v, mask_v, out_v,
                a_sem0, a_sem1, a_sem2, a_sem3,
                o_sem0, o_sem1, o_sem2, o_sem3, m_sem0, m_sem1):
    wid = lax.axis_index("s") * NC + lax.axis_index("c")
    base = wid * ROWS_PER_W
    kb = wid * (ROWS_PER_W // 4)
    lane = lax.iota(jnp.int32, L)
    last_splat = jnp.full((L,), L - 1, jnp.int32)
    a_sems = (a_sem0, a_sem1, a_sem2, a_sem3)
    o_sems = (o_sem0, o_sem1, o_sem2, o_sem3)
    m_sems = (m_sem0, m_sem1)

    def start_attn(sub, r):
        pltpu.async_copy(attn_hbm.at[r], attn_v.at[pl.ds(sub * S, S)],
                         a_sems[sub])

    def wait_attn(sub):
        pltpu.make_async_copy(attn_hbm.at[0], attn_v.at[pl.ds(sub * S, S)],
                              a_sems[sub]).wait()

    def start_mask(mq, k):
        pltpu.async_copy(mask_hbm.at[k], mask_v.at[pl.ds(mq * S, S)],
                         m_sems[mq])

    def wait_mask(mq):
        pltpu.make_async_copy(mask_hbm.at[0], mask_v.at[pl.ds(mq * S, S)],
                              m_sems[mq]).wait()

    def wait_out(sub):
        pltpu.make_async_copy(out_hbm.at[0], out_v.at[pl.ds(sub * S, S)],
                              o_sems[sub]).wait()

    for sub in range(4):
        start_attn(sub, base + sub)
    for mq in range(2):
        start_mask(mq, kb + mq)

    def do_oct(gp, _):
        for mq in range(2):
            wait_mask(mq)
            for sub in range(4):
                r = base + gp * 8 + mq * 4 + sub
                wait_attn(sub)
                if mq == 0:
                    @pl.when(gp > 0)
                    def _():
                        wait_out(sub)
                else:
                    wait_out(sub)

                def group(q, carry, _mq=mq, _sub=sub):
                    for j in range(UNROLL):
                        c = q * UNROLL + j
                        a = attn_v[pl.ds(_sub * S + c * L, L)]
                        w = mask_v[pl.ds(_mq * S + c * L, L)]
                        dry = ((w >> (8 * _sub)) & 1) == 0
                        didx = jnp.where(dry, lane, jnp.int32(-1))
                        mx = plsc.cummax(didx)
                        gval = _gather16(a, jnp.maximum(mx, 0))
                        res = jnp.where(mx >= 0, gval, carry)
                        out_v[pl.ds(_sub * S + c * L, L)] = res
                        carry = _gather16(res, last_splat)
                    return carry

                lax.fori_loop(0, GROUPS, group, jnp.zeros((L,), jnp.float32))
                pltpu.async_copy(out_v.at[pl.ds(sub * S, S)], out_hbm.at[r],
                                 o_sems[sub])
                if mq == 0:
                    start_attn(sub, r + 4)
                else:
                    @pl.when(gp < (ROWS_PER_W // 8) - 1)
                    def _():
                        start_attn(sub, r + 4)

            @pl.when(gp < (ROWS_PER_W // 8) - 1)
            def _(_mq=mq):
                start_mask(_mq, kb + gp * 2 + _mq + 2)
        return 0

    lax.fori_loop(0, ROWS_PER_W // 8, do_oct, 0)
    for sub in range(4):
        wait_out(sub)


def kernel(input_attenuation, input_wet_dry):
    wd = input_wet_dry
    packed = (wd[0::4].astype(jnp.int32)
              | (wd[1::4].astype(jnp.int32) << 8)
              | (wd[2::4].astype(jnp.int32) << 16)
              | (wd[3::4].astype(jnp.int32) << 24))
    packed = jnp.where(lax.iota(jnp.int32, S)[None, :] > 0, packed, 0)
    mesh = plsc.VectorSubcoreMesh(core_axis_name="c", subcore_axis_name="s")
    f = pl.kernel(
        _ffill_body,
        mesh=mesh,
        compiler_params=pltpu.CompilerParams(needs_layout_passes=False),
        out_type=jax.ShapeDtypeStruct((N, S), jnp.float32),
        scratch_types=[
            pltpu.VMEM((4 * S,), jnp.float32),
            pltpu.VMEM((2 * S,), jnp.int32),
            pltpu.VMEM((4 * S,), jnp.float32),
            pltpu.SemaphoreType.DMA,
            pltpu.SemaphoreType.DMA,
            pltpu.SemaphoreType.DMA,
            pltpu.SemaphoreType.DMA,
            pltpu.SemaphoreType.DMA,
            pltpu.SemaphoreType.DMA,
            pltpu.SemaphoreType.DMA,
            pltpu.SemaphoreType.DMA,
            pltpu.SemaphoreType.DMA,
            pltpu.SemaphoreType.DMA,
        ],
    )
    return f(input_attenuation, packed)


# speedup vs baseline: 4.1925x; 4.1925x over previous
import jax
import jax.numpy as jnp
from jax import lax
from jax.experimental import pallas as pl
from jax.experimental.pallas import tpu as pltpu
from jax.experimental.pallas import tpu_sc as plsc

N, S = 1024, 8192
L = 16
NC, NS = 2, 16
NW = NC * NS
ROWS_PER_W = N // NW
NBUF = 2
NSEG = 4
SEG = S // NSEG
UNROLL = 8
SEG_GROUPS = SEG // (UNROLL * L)

_GDN = lax.GatherDimensionNumbers(
    offset_dims=(), collapsed_slice_dims=(0,), start_index_map=(0,))


def _gather16(v, idx):
    return lax.gather(v, idx[:, None], _GDN, slice_sizes=(1,),
                      mode=lax.GatherScatterMode.PROMISE_IN_BOUNDS)


def _ffill_body(attn_hbm, mask_hbm, out_hbm, attn_v, mask_v, out_v,
                in_sem0, in_sem1, out_sem0, out_sem1):
    wid = lax.axis_index("s") * NC + lax.axis_index("c")
    base = wid * ROWS_PER_W
    lane = lax.iota(jnp.int32, L)
    last_splat = jnp.full((L,), L - 1, jnp.int32)
    in_sems = (in_sem0, in_sem1)
    out_sems = (out_sem0, out_sem1)

    def start_in(b, r):
        pltpu.async_copy(attn_hbm.at[r], attn_v.at[pl.ds(b * S, S)],
                         in_sems[b])
        pltpu.async_copy(mask_hbm.at[r], mask_v.at[pl.ds(b * SEG, SEG)],
                         in_sems[b])

    def wait_in(b):
        pltpu.make_async_copy(attn_hbm.at[0], attn_v.at[pl.ds(b * S, S)],
                              in_sems[b]).wait()
        pltpu.make_async_copy(mask_hbm.at[0], mask_v.at[pl.ds(b * SEG, SEG)],
                              in_sems[b]).wait()

    for b in range(NBUF):
        start_in(b, base + b)

    def do_pair(g, _):
        for b in range(NBUF):
            r = base + g * NBUF + b
            wait_in(b)

            @pl.when(g > 0)
            def _():
                pltpu.make_async_copy(out_hbm.at[0],
                                      out_v.at[pl.ds(b * S, S)],
                                      out_sems[b]).wait()

            carry = jnp.zeros((L,), jnp.float32)
            for seg in range(NSEG):
                def group(q, carry, _seg=seg):
                    for j in range(UNROLL):
                        t = (q * UNROLL + j) * L
                        off = b * S + _seg * SEG + t
                        a = attn_v[pl.ds(off, L)]
                        w = mask_v[pl.ds(b * SEG + t, L)]
                        dry = ((w >> (8 * _seg)) & 1) == 0
                        didx = jnp.where(dry, lane, jnp.int32(-1))
                        mx = plsc.cummax(didx)
                        gval = _gather16(a, jnp.maximum(mx, 0))
                        res = jnp.where(mx >= 0, gval, carry)
                        out_v[pl.ds(off, L)] = res
                        carry = _gather16(res, last_splat)
                    return carry

                carry = lax.fori_loop(0, SEG_GROUPS, group, carry)

            pltpu.async_copy(out_v.at[pl.ds(b * S, S)], out_hbm.at[r],
                             out_sems[b])

            @pl.when(g + 1 < ROWS_PER_W // NBUF)
            def _():
                start_in(b, r + NBUF)
        return 0

    lax.fori_loop(0, ROWS_PER_W // NBUF, do_pair, 0)
    for b in range(NBUF):
        pltpu.make_async_copy(out_hbm.at[0], out_v.at[pl.ds(b * S, S)],
                              out_sems[b]).wait()


def kernel(input_attenuation, input_wet_dry):
    wd = input_wet_dry
    packed = (wd[:, 0 * SEG:1 * SEG].astype(jnp.int32)
              | (wd[:, 1 * SEG:2 * SEG].astype(jnp.int32) << 8)
              | (wd[:, 2 * SEG:3 * SEG].astype(jnp.int32) << 16)
              | (wd[:, 3 * SEG:4 * SEG].astype(jnp.int32) << 24))
    packed = jnp.where(lax.iota(jnp.int32, SEG)[None, :] > 0, packed,
                       packed & ~jnp.int32(1))
    mesh = plsc.VectorSubcoreMesh(core_axis_name="c", subcore_axis_name="s")
    f = pl.kernel(
        _ffill_body,
        mesh=mesh,
        compiler_params=pltpu.CompilerParams(needs_layout_passes=False),
        out_type=jax.ShapeDtypeStruct((N, S), jnp.float32),
        scratch_types=[
            pltpu.VMEM((NBUF * S,), jnp.float32),
            pltpu.VMEM((NBUF * SEG,), jnp.int32),
            pltpu.VMEM((NBUF * S,), jnp.float32),
            pltpu.SemaphoreType.DMA,
            pltpu.SemaphoreType.DMA,
            pltpu.SemaphoreType.DMA,
            pltpu.SemaphoreType.DMA,
        ],
    )
    return f(input_attenuation, packed)
